# Initial kernel scaffold; baseline (speedup 1.0000x reference)
import jax
import jax.numpy as jnp
from jax.experimental import pallas as pl


def kernel(input_ids, table, offsets):
    raise NotImplementedError("write your pallas kernel here")


# speedup vs baseline: 2.1848x; 2.1848x over previous
import functools

import jax
import jax.numpy as jnp
from jax import lax
from jax.experimental import pallas as pl
from jax.experimental.pallas import tpu as pltpu
from jax.experimental.pallas import tpu_sc as plsc

B, T, H, D = 1024, 200, 8, 32
N = B * T * H

NC, NS, L = 2, 16, 16
NW = NC * NS
PER_W = N // NW
C = 128
S = PER_W // C
K = 8
OUTER = S // K


def _sc_gather(idx_hbm, table_hbm, off_hbm):
    mesh = plsc.VectorSubcoreMesh(core_axis_name="c", subcore_axis_name="s")

    @functools.partial(
        pl.kernel,
        out_type=jax.ShapeDtypeStruct((N, D), jnp.float32),
        mesh=mesh,
        compiler_params=pltpu.CompilerParams(use_tc_tiling_on_sc=False),
        scratch_types=[
            pltpu.VMEM((S, C), jnp.int32),
            pltpu.VMEM((16,), jnp.int32),
            pltpu.VMEM((K * C, D), jnp.float32),
            pltpu.SemaphoreType.DMA,
            pltpu.SemaphoreType.DMA,
        ],
    )
    def k(idx_ref, table_ref, off_ref, out_ref, idx_v, off_v, rows_v, gsem, osem):
        wid = lax.axis_index("s") * NC + lax.axis_index("c")
        base = wid * PER_W

        pltpu.sync_copy(off_ref, off_v)
        pltpu.sync_copy(idx_ref.at[wid], idx_v)
        off = off_v[...]

        def add_body(s, _):
            for i in range(C // L):
                sl = pl.ds(i * L, L)
                idx_v[s, sl] = idx_v[s, sl] + off
            return _

        lax.fori_loop(0, S, add_body, None, unroll=False)

        def outer_body(j, _):
            copies = []
            for b in range(K):
                cp = pltpu.async_copy(
                    table_ref.at[idx_v.at[j * K + b]],
                    rows_v.at[pl.ds(b * C, C)],
                    gsem,
                )
                copies.append(cp)
            for cp in copies:
                cp.wait()
            pltpu.sync_copy(rows_v, out_ref.at[pl.ds(base + j * (K * C), K * C)])
            return _

        lax.fori_loop(0, OUTER, outer_body, None, unroll=False)

    return k(idx_hbm, table_hbm, off_hbm)


def kernel(input_ids, table, offsets):
    idx3 = input_ids.reshape(NW, S, C)
    off16 = jnp.concatenate([offsets, offsets])
    out = _sc_gather(idx3, table, off16)
    return out.reshape(B, T, H, D)


# speedup vs baseline: 3.5193x; 1.6108x over previous
import functools

import jax
import jax.numpy as jnp
from jax import lax
from jax.experimental import pallas as pl
from jax.experimental.pallas import tpu as pltpu
from jax.experimental.pallas import tpu_sc as plsc

B, T, H, D = 1024, 200, 8, 32
N = B * T * H

NC, NS, L = 2, 16, 16
NW = NC * NS
PER_W = N // NW
C = 128
S = PER_W // C
K = 8
OUTER = S // K


def _sc_gather(idx_hbm, table_hbm, off_hbm):
    mesh = plsc.VectorSubcoreMesh(core_axis_name="c", subcore_axis_name="s")

    @functools.partial(
        pl.kernel,
        out_type=jax.ShapeDtypeStruct((N, 128), jnp.float32),
        mesh=mesh,
        compiler_params=pltpu.CompilerParams(use_tc_tiling_on_sc=False),
        scratch_types=[
            pltpu.VMEM((S, C), jnp.int32),
            pltpu.VMEM((16,), jnp.int32),
            pltpu.VMEM((K * C, D), jnp.float32),
            pltpu.SemaphoreType.DMA,
            pltpu.SemaphoreType.DMA,
        ],
    )
    def k(idx_ref, table_ref, off_ref, out_ref, idx_v, off_v, rows_v, gsem, osem):
        wid = lax.axis_index("s") * NC + lax.axis_index("c")
        base = wid * PER_W

        pltpu.sync_copy(off_ref, off_v)
        pltpu.sync_copy(idx_ref.at[wid], idx_v)
        off = off_v[...]

        def add_body(s, _):
            for i in range(C // L):
                sl = pl.ds(i * L, L)
                idx_v[s, sl] = idx_v[s, sl] + off
            return _

        lax.fori_loop(0, S, add_body, None, unroll=False)

        def outer_body(j, _):
            copies = []
            for b in range(K):
                cp = pltpu.async_copy(
                    table_ref.at[idx_v.at[j * K + b]],
                    rows_v.at[pl.ds(b * C, C)],
                    gsem,
                )
                copies.append(cp)
            for cp in copies:
                cp.wait()
            pltpu.sync_copy(rows_v,
                            out_ref.at[pl.ds(base + j * (K * C), K * C), pl.ds(0, D)])
            return _

        lax.fori_loop(0, OUTER, outer_body, None, unroll=False)

    return k(idx_hbm, table_hbm, off_hbm)


def kernel(input_ids, table, offsets):
    idx3 = input_ids.reshape(NW, S, C)
    off16 = jnp.concatenate([offsets, offsets])
    out = _sc_gather(idx3, table, off16)
    return out[:, :D].reshape(B, T, H, D)
